# baseline (speedup 1.0000x reference)
import functools

import jax
import jax.numpy as jnp
from jax import lax
from jax.experimental import pallas as pl
from jax.experimental.pallas import tpu as pltpu
from jax.experimental.pallas import tpu_sc as plsc

N_CORES = 2
N_SUBCORES = 16
NW = N_CORES * N_SUBCORES
BATCH = 16384
DIM = 64
B_PER_W = BATCH // NW
CHUNK = 128
N_CHUNKS = B_PER_W // CHUNK
GROUPS = CHUNK // 16

def _rsqrt16(x):
    x = jnp.maximum(x, jnp.float32(1e-24))
    i = plsc.bitcast(x, jnp.int32)
    i = jnp.int32(0x5F3759DF) - lax.shift_right_arithmetic(i, jnp.int32(1))
    y = plsc.bitcast(i, jnp.float32)
    half = jnp.float32(0.5) * x
    for _ in range(3):
        y = y * (jnp.float32(1.5) - half * y * y)
    return y


def _sc_body(h_hbm, r_hbm, t_hbm, ent_hbm, rel_hbm, out_hbm,
             idx_h, idx_r, idx_t, rows_h, rows_r, rows_t, out_v, sem):
    wid = lax.axis_index("s") * N_CORES + lax.axis_index("c")
    base = wid * B_PER_W

    for c in range(N_CHUNKS):
        off = base + c * CHUNK
        pltpu.sync_copy(h_hbm.at[pl.ds(off, CHUNK)], idx_h.at[c])
        pltpu.sync_copy(r_hbm.at[pl.ds(off, CHUNK)], idx_r.at[c])
        pltpu.sync_copy(t_hbm.at[pl.ds(off, CHUNK)], idx_t.at[c])

    iota16 = lax.iota(jnp.int32, 16)

    for c in range(N_CHUNKS):
        cp_h = pltpu.async_copy(ent_hbm.at[idx_h.at[c]], rows_h, sem)
        cp_r = pltpu.async_copy(rel_hbm.at[idx_r.at[c]], rows_r, sem)
        cp_t = pltpu.async_copy(ent_hbm.at[idx_t.at[c]], rows_t, sem)
        cp_h.wait()
        cp_r.wait()
        cp_t.wait()


        def group_body(g, _, c=c):
            row16 = iota16 + g * 16

            def col_body(j, accs):
                hh, tt, rr, hr, ht, rt = accs
                col = lax.broadcast(j, (16,))
                hv = plsc.load_gather(rows_h, [row16, col])
                rv = plsc.load_gather(rows_r, [row16, col])
                tv = plsc.load_gather(rows_t, [row16, col])
                hh = hh + hv * hv
                tt = tt + tv * tv
                rr = rr + rv * rv
                hr = hr + hv * rv
                ht = ht + hv * tv
                rt = rt + rv * tv
                return (hh, tt, rr, hr, ht, rt)

            zero = jnp.zeros((16,), jnp.float32)
            hh, tt, rr, hr, ht, rt = lax.fori_loop(
                0, DIM, col_body, (zero, zero, zero, zero, zero, zero))

            inh = _rsqrt16(hh)
            int_ = _rsqrt16(tt)
            s2 = (hh * inh * inh + tt * int_ * int_ + rr
                  + jnp.float32(2.0) * (hr * inh - ht * inh * int_ - rt * int_))
            s2 = jnp.maximum(s2, jnp.float32(0.0))
            out16 = -(s2 * _rsqrt16(s2))
            out_v[pl.ds(c * CHUNK + g * 16, 16)] = out16
            return 0

        lax.fori_loop(0, GROUPS, group_body, 0)

    pltpu.sync_copy(out_v, out_hbm.at[pl.ds(base, B_PER_W)])


@functools.partial(jax.jit, static_argnames=())
def kernel(h, r, t, ent_emb, rel_emb):
    mesh = plsc.VectorSubcoreMesh(
        core_axis_name="c", subcore_axis_name="s",
        num_cores=N_CORES, num_subcores=N_SUBCORES)
    run = pl.kernel(
        _sc_body,
        out_type=jax.ShapeDtypeStruct((BATCH,), jnp.float32),
        mesh=mesh,
        compiler_params=pltpu.CompilerParams(needs_layout_passes=False, use_tc_tiling_on_sc=False),
        scratch_types=[
            pltpu.VMEM((N_CHUNKS, CHUNK), jnp.int32),
            pltpu.VMEM((N_CHUNKS, CHUNK), jnp.int32),
            pltpu.VMEM((N_CHUNKS, CHUNK), jnp.int32),
            pltpu.VMEM((CHUNK, DIM), jnp.float32),
            pltpu.VMEM((CHUNK, DIM), jnp.float32),
            pltpu.VMEM((CHUNK, DIM), jnp.float32),
            pltpu.VMEM((B_PER_W,), jnp.float32),
            pltpu.SemaphoreType.DMA,
        ],
    )
    return run(h, r, t, ent_emb, rel_emb)
